# baseline (speedup 1.0000x reference)
import functools

import jax
import jax.numpy as jnp
from jax import lax
from jax.experimental import pallas as pl
from jax.experimental.pallas import tpu as pltpu
from jax.experimental.pallas import tpu_sc as plsc

TOPK = 8
E = 64
SCALE = 2.5
T = 32768
NW = 32
RPW = T // NW


def _router_body(logits_hbm, bias_hbm, w_hbm, idx_hbm, cnt_hbm,
                 in_v, w_v, i_v, cnt_v, bias_v):
    c = lax.axis_index("c")
    s = lax.axis_index("s")
    wid = s * 2 + c
    base = wid * RPW

    pltpu.sync_copy(bias_hbm, bias_v)
    pltpu.sync_copy(logits_hbm.at[pl.ds(base * E, RPW * E)], in_v)

    lane = lax.iota(jnp.int32, 16)
    lt8 = lane < 8
    zeros16 = jnp.zeros((16,), jnp.int32)
    for j in range(4):
        cnt_v[pl.ds(16 * j, 16)] = zeros16
    bias_regs = [bias_v[pl.ds(16 * j, 16)] for j in range(4)]
    ones16 = jnp.ones((16,), jnp.int32)

    def merge(k0, v0, k1, v1):
        k1r = lax.rev(k1, (0,))
        v1r = lax.rev(v1, (0,))
        km = jnp.where(lt8, k0, k1r)
        vm = jnp.where(lt8, v0, v1r)
        return plsc.sort_key_val(km, vm, descending=True)

    def take(x, idx):
        return x.at[idx].get(mode="promise_in_bounds")

    lane_and7 = jnp.bitwise_and(lane, 7)

    def merge_tree(r):
        sk = []
        sv = []
        for j in range(4):
            x = in_v[pl.ds(r * E + 16 * j, 16)]
            sg = 1.0 / (1.0 + jnp.exp(-x))
            b = sg + bias_regs[j]
            k_s, v_s = plsc.sort_key_val(b, lane + 16 * j, descending=True)
            sk.append(k_s)
            sv.append(v_s)
        ka, va = merge(sk[0], sv[0], sk[1], sv[1])
        kb, vb = merge(sk[2], sv[2], sk[3], sv[3])
        return merge(ka, va, kb, vb)

    @plsc.parallel_loop(0, RPW // 2, unroll=2)
    def do_pair(p):
        rA = 2 * p
        kA, vA = merge_tree(rA)
        kB, vB = merge_tree(rA + 1)
        kP = jnp.where(lt8, kA, take(kB, lane_and7))
        vP = jnp.where(lt8, vA, take(vB, lane_and7))
        bg = plsc.load_gather(bias_v, [vP])
        w = kP - bg
        s = w
        for m in (1, 2, 4):
            s = s + take(s, jnp.bitwise_xor(lane, m))
        wn = (w * SCALE) / (s + 1e-20)
        w_v[pl.ds(pl.multiple_of(16 * p, 8), 16)] = wn
        i_v[pl.ds(pl.multiple_of(16 * p, 8), 16)] = vP
        plsc.addupdate_scatter(cnt_v, [vP], ones16, mask=lt8)
        plsc.addupdate_scatter(cnt_v, [vP], ones16, mask=jnp.logical_not(lt8))

    pltpu.sync_copy(w_v.at[pl.ds(0, RPW * TOPK)],
                    w_hbm.at[pl.ds(base * TOPK, RPW * TOPK)])
    pltpu.sync_copy(i_v.at[pl.ds(0, RPW * TOPK)],
                    idx_hbm.at[pl.ds(base * TOPK, RPW * TOPK)])
    pltpu.sync_copy(cnt_v, cnt_hbm.at[pl.ds(wid * E, E)])


_router = pl.kernel(
    _router_body,
    out_type=(
        jax.ShapeDtypeStruct((T * TOPK,), jnp.float32),
        jax.ShapeDtypeStruct((T * TOPK,), jnp.int32),
        jax.ShapeDtypeStruct((NW * E,), jnp.int32),
    ),
    mesh=plsc.VectorSubcoreMesh(core_axis_name="c", subcore_axis_name="s"),
    compiler_params=pltpu.CompilerParams(needs_layout_passes=False),
    scratch_types=(
        pltpu.VMEM((RPW * E,), jnp.float32),
        pltpu.VMEM((RPW * TOPK,), jnp.float32),
        pltpu.VMEM((RPW * TOPK,), jnp.int32),
        pltpu.VMEM((E,), jnp.int32),
        pltpu.VMEM((E,), jnp.float32),
    ),
)


def kernel(logits, e_score_correction_bias):
    w_flat, i_flat, cnt_part = _router(logits.reshape(-1),
                                       e_score_correction_bias)
    topk_weight = w_flat.reshape(T, TOPK)
    topk_idx = i_flat.reshape(T, TOPK)
    return (logits, topk_weight, topk_idx, cnt_part)


# speedup vs baseline: 1.2204x; 1.2204x over previous
import functools

import jax
import jax.numpy as jnp
from jax import lax
from jax.experimental import pallas as pl
from jax.experimental.pallas import tpu as pltpu
from jax.experimental.pallas import tpu_sc as plsc

TOPK = 8
E = 64
SCALE = 2.5
T = 32768
NW = 32
RPW = T // NW
CH = 64
NCH = RPW // CH


def _router_body(logits_hbm, bias_hbm, w_hbm, idx_hbm, cnt_hbm,
                 in_v, w_ca, w_cb, i_ca, i_cb, cnt_v, bias_v,
                 sem_wa, sem_ia, sem_wb, sem_ib):
    c = lax.axis_index("c")
    s = lax.axis_index("s")
    wid = s * 2 + c
    base = wid * RPW

    pltpu.sync_copy(bias_hbm, bias_v)
    pltpu.sync_copy(logits_hbm.at[pl.ds(base * E, RPW * E)], in_v)

    lane = lax.iota(jnp.int32, 16)
    lt8 = lane < 8
    zeros16 = jnp.zeros((16,), jnp.int32)
    for j in range(4):
        cnt_v[pl.ds(16 * j, 16)] = zeros16
    bias_regs = [bias_v[pl.ds(16 * j, 16)] for j in range(4)]
    ones16 = jnp.ones((16,), jnp.int32)

    def merge(k0, v0, k1, v1):
        k1r = lax.rev(k1, (0,))
        v1r = lax.rev(v1, (0,))
        km = jnp.where(lt8, k0, k1r)
        vm = jnp.where(lt8, v0, v1r)
        return plsc.sort_key_val(km, vm, descending=True)

    def take(x, idx):
        return x.at[idx].get(mode="promise_in_bounds")

    lane_and7 = jnp.bitwise_and(lane, 7)

    def merge_tree(r):
        sk = []
        sv = []
        for j in range(4):
            x = in_v[pl.ds(r * E + 16 * j, 16)]
            sg = 1.0 / (1.0 + jnp.exp(-x))
            b = sg + bias_regs[j]
            k_s, v_s = plsc.sort_key_val(b, lane + 16 * j, descending=True)
            sk.append(k_s)
            sv.append(v_s)
        ka, va = merge(sk[0], sv[0], sk[1], sv[1])
        kb, vb = merge(sk[2], sv[2], sk[3], sv[3])
        return merge(ka, va, kb, vb)

    shr3 = lax.shift_right_logical(lane, 3)

    def do_pair(p, pbase, w_c, i_c):
        rA = 2 * p
        kA, vA = merge_tree(rA)
        kB, vB = merge_tree(rA + 1)
        kP = jnp.where(lt8, kA, take(kB, lane_and7))
        vP = jnp.where(lt8, vA, take(vB, lane_and7))
        bg = plsc.load_gather(bias_v, [vP])
        w = kP - bg
        s = w
        for m in (1, 2, 4):
            s = s + take(s, jnp.bitwise_xor(lane, m))
        wn = (w * SCALE) / (s + 1e-20)
        rowv = 2 * (p - pbase) + shr3
        plsc.store_scatter(w_c, [rowv, lane_and7], wn)
        plsc.store_scatter(i_c, [rowv, lane_and7], vP)
        plsc.addupdate_scatter(cnt_v, [vP], ones16, mask=lt8)
        plsc.addupdate_scatter(cnt_v, [vP], ones16, mask=jnp.logical_not(lt8))

    PPC = CH // 2

    def run_chunk(ch, ch2, w_c, i_c, sem_w, sem_i):
        @pl.when(ch2 > 0)
        def _():
            pltpu.make_async_copy(
                w_c, w_hbm.at[pl.ds(base, CH), :], sem_w).wait()
            pltpu.make_async_copy(
                i_c, idx_hbm.at[pl.ds(base, CH), :], sem_i).wait()

        pbase = ch * PPC

        @plsc.parallel_loop(0, PPC, unroll=2)
        def _(p):
            do_pair(pbase + p, pbase, w_c, i_c)

        row0 = base + ch * CH
        pltpu.async_copy(w_c, w_hbm.at[pl.ds(row0, CH), :], sem_w)
        pltpu.async_copy(i_c, idx_hbm.at[pl.ds(row0, CH), :], sem_i)

    def outer(ch2, carry):
        run_chunk(2 * ch2, ch2, w_ca, i_ca, sem_wa, sem_ia)
        run_chunk(2 * ch2 + 1, ch2, w_cb, i_cb, sem_wb, sem_ib)
        return carry

    lax.fori_loop(0, NCH // 2, outer, 0)

    for w_c, i_c, sem_w, sem_i in ((w_ca, i_ca, sem_wa, sem_ia),
                                   (w_cb, i_cb, sem_wb, sem_ib)):
        pltpu.make_async_copy(w_c, w_hbm.at[pl.ds(base, CH), :], sem_w).wait()
        pltpu.make_async_copy(i_c, idx_hbm.at[pl.ds(base, CH), :], sem_i).wait()

    pltpu.sync_copy(cnt_v, cnt_hbm.at[pl.ds(wid * E, E)])


_router = pl.kernel(
    _router_body,
    out_type=(
        jax.ShapeDtypeStruct((T, TOPK), jnp.float32),
        jax.ShapeDtypeStruct((T, TOPK), jnp.int32),
        jax.ShapeDtypeStruct((NW * E,), jnp.int32),
    ),
    mesh=plsc.VectorSubcoreMesh(core_axis_name="c", subcore_axis_name="s"),
    compiler_params=pltpu.CompilerParams(needs_layout_passes=False,
                                         use_tc_tiling_on_sc=True),
    scratch_types=(
        pltpu.VMEM((RPW * E,), jnp.float32),
        pltpu.VMEM((CH, TOPK), jnp.float32),
        pltpu.VMEM((CH, TOPK), jnp.float32),
        pltpu.VMEM((CH, TOPK), jnp.int32),
        pltpu.VMEM((CH, TOPK), jnp.int32),
        pltpu.VMEM((E,), jnp.int32),
        pltpu.VMEM((E,), jnp.float32),
        pltpu.SemaphoreType.DMA,
        pltpu.SemaphoreType.DMA,
        pltpu.SemaphoreType.DMA,
        pltpu.SemaphoreType.DMA,
    ),
)


def kernel(logits, e_score_correction_bias):
    topk_weight, topk_idx, cnt_part = _router(logits.reshape(-1),
                                              e_score_correction_bias)
    tokens_per_expert = cnt_part.reshape(NW, E).sum(axis=0)
    return (logits, topk_weight, topk_idx, tokens_per_expert)


# speedup vs baseline: 1.3526x; 1.1083x over previous
import functools

import jax
import jax.numpy as jnp
from jax import lax
from jax.experimental import pallas as pl
from jax.experimental.pallas import tpu as pltpu
from jax.experimental.pallas import tpu_sc as plsc

TOPK = 8
E = 64
SCALE = 2.5
T = 32768
NW = 32
RPW = T // NW
CH = 64
NCH = RPW // CH


def _router_body(logits_hbm, bias_hbm, w_hbm, idx_hbm, cnt_hbm,
                 in_ca, in_cb, w_ca, w_cb, i_ca, i_cb, cnt_v, bias_v,
                 sem_na, sem_nb, sem_wa, sem_ia, sem_wb, sem_ib):
    c = lax.axis_index("c")
    s = lax.axis_index("s")
    wid = s * 2 + c
    base = wid * RPW

    pltpu.sync_copy(bias_hbm, bias_v)
    pltpu.async_copy(logits_hbm.at[pl.ds(base, CH), :], in_ca, sem_na)
    pltpu.async_copy(logits_hbm.at[pl.ds(base + CH, CH), :], in_cb, sem_nb)

    lane = lax.iota(jnp.int32, 16)
    lt8 = lane < 8
    zeros16 = jnp.zeros((16,), jnp.int32)
    for j in range(4):
        cnt_v[pl.ds(16 * j, 16)] = zeros16
    bias_regs = [bias_v[pl.ds(16 * j, 16)] for j in range(4)]
    ones16 = jnp.ones((16,), jnp.int32)

    def merge(k0, v0, k1, v1):
        k1r = lax.rev(k1, (0,))
        v1r = lax.rev(v1, (0,))
        km = jnp.where(lt8, k0, k1r)
        vm = jnp.where(lt8, v0, v1r)
        return plsc.sort_key_val(km, vm, descending=True)

    def take(x, idx):
        return x.at[idx].get(mode="promise_in_bounds")

    lane_and7 = jnp.bitwise_and(lane, 7)

    def merge_tree(in_c, r):
        sk = []
        sv = []
        for j in range(4):
            x = in_c[r, pl.ds(16 * j, 16)]
            sg = 1.0 / (1.0 + jnp.exp(-x))
            b = sg + bias_regs[j]
            k_s, v_s = plsc.sort_key_val(b, lane + 16 * j, descending=True)
            sk.append(k_s)
            sv.append(v_s)
        ka, va = merge(sk[0], sv[0], sk[1], sv[1])
        kb, vb = merge(sk[2], sv[2], sk[3], sv[3])
        return merge(ka, va, kb, vb)

    shr3 = lax.shift_right_logical(lane, 3)

    def do_pair(p, pbase, in_c, w_c, i_c):
        rA = 2 * (p - pbase)
        kA, vA = merge_tree(in_c, rA)
        kB, vB = merge_tree(in_c, rA + 1)
        kP = jnp.where(lt8, kA, take(kB, lane_and7))
        vP = jnp.where(lt8, vA, take(vB, lane_and7))
        bg = plsc.load_gather(bias_v, [vP])
        w = kP - bg
        s = w
        for m in (1, 2, 4):
            s = s + take(s, jnp.bitwise_xor(lane, m))
        wn = (w * SCALE) / (s + 1e-20)
        rowv = 2 * (p - pbase) + shr3
        plsc.store_scatter(w_c, [rowv, lane_and7], wn)
        plsc.store_scatter(i_c, [rowv, lane_and7], vP)
        plsc.addupdate_scatter(cnt_v, [vP], ones16, mask=lt8)
        plsc.addupdate_scatter(cnt_v, [vP], ones16, mask=jnp.logical_not(lt8))

    PPC = CH // 2

    def run_chunk(ch, ch2, in_c, w_c, i_c, sem_n, sem_w, sem_i):
        pltpu.make_async_copy(
            logits_hbm.at[pl.ds(base, CH), :], in_c, sem_n).wait()

        @pl.when(ch2 > 0)
        def _():
            pltpu.make_async_copy(
                w_c, w_hbm.at[pl.ds(base, CH), :], sem_w).wait()
            pltpu.make_async_copy(
                i_c, idx_hbm.at[pl.ds(base, CH), :], sem_i).wait()

        pbase = ch * PPC

        @plsc.parallel_loop(0, PPC, unroll=2)
        def _(p):
            do_pair(pbase + p, pbase, in_c, w_c, i_c)

        row0 = base + ch * CH
        pltpu.async_copy(w_c, w_hbm.at[pl.ds(row0, CH), :], sem_w)
        pltpu.async_copy(i_c, idx_hbm.at[pl.ds(row0, CH), :], sem_i)

        @pl.when(ch + 2 < NCH)
        def _():
            rown = base + (ch + 2) * CH
            pltpu.async_copy(logits_hbm.at[pl.ds(rown, CH), :], in_c, sem_n)

    def outer(ch2, carry):
        run_chunk(2 * ch2, ch2, in_ca, w_ca, i_ca, sem_na, sem_wa, sem_ia)
        run_chunk(2 * ch2 + 1, ch2, in_cb, w_cb, i_cb, sem_nb, sem_wb, sem_ib)
        return carry

    lax.fori_loop(0, NCH // 2, outer, 0)

    for w_c, i_c, sem_w, sem_i in ((w_ca, i_ca, sem_wa, sem_ia),
                                   (w_cb, i_cb, sem_wb, sem_ib)):
        pltpu.make_async_copy(w_c, w_hbm.at[pl.ds(base, CH), :], sem_w).wait()
        pltpu.make_async_copy(i_c, idx_hbm.at[pl.ds(base, CH), :], sem_i).wait()

    pltpu.sync_copy(cnt_v, cnt_hbm.at[pl.ds(wid * E, E)])


_router = pl.kernel(
    _router_body,
    out_type=(
        jax.ShapeDtypeStruct((T, TOPK), jnp.float32),
        jax.ShapeDtypeStruct((T, TOPK), jnp.int32),
        jax.ShapeDtypeStruct((NW * E,), jnp.int32),
    ),
    mesh=plsc.VectorSubcoreMesh(core_axis_name="c", subcore_axis_name="s"),
    compiler_params=pltpu.CompilerParams(needs_layout_passes=False,
                                         use_tc_tiling_on_sc=True),
    scratch_types=(
        pltpu.VMEM((CH, E), jnp.float32),
        pltpu.VMEM((CH, E), jnp.float32),
        pltpu.VMEM((CH, TOPK), jnp.float32),
        pltpu.VMEM((CH, TOPK), jnp.float32),
        pltpu.VMEM((CH, TOPK), jnp.int32),
        pltpu.VMEM((CH, TOPK), jnp.int32),
        pltpu.VMEM((E,), jnp.int32),
        pltpu.VMEM((E,), jnp.float32),
        pltpu.SemaphoreType.DMA,
        pltpu.SemaphoreType.DMA,
        pltpu.SemaphoreType.DMA,
        pltpu.SemaphoreType.DMA,
        pltpu.SemaphoreType.DMA,
        pltpu.SemaphoreType.DMA,
    ),
)


def kernel(logits, e_score_correction_bias):
    topk_weight, topk_idx, cnt_part = _router(logits,
                                              e_score_correction_bias)
    tokens_per_expert = cnt_part.reshape(NW, E).sum(axis=0)
    return (logits, topk_weight, topk_idx, tokens_per_expert)


# speedup vs baseline: 1.3560x; 1.0025x over previous
import functools

import jax
import jax.numpy as jnp
from jax import lax
from jax.experimental import pallas as pl
from jax.experimental.pallas import tpu as pltpu
from jax.experimental.pallas import tpu_sc as plsc

TOPK = 8
E = 64
SCALE = 2.5
T = 32768
NW = 32
RPW = T // NW
CH = 128
NCH = RPW // CH


def _router_body(logits_hbm, bias_hbm, w_hbm, idx_hbm, cnt_hbm,
                 in_ca, in_cb, w_ca, w_cb, i_ca, i_cb, cnt_v, bias_v,
                 sem_na, sem_nb, sem_wa, sem_ia, sem_wb, sem_ib):
    c = lax.axis_index("c")
    s = lax.axis_index("s")
    wid = s * 2 + c
    base = wid * RPW

    pltpu.sync_copy(bias_hbm, bias_v)
    pltpu.async_copy(logits_hbm.at[pl.ds(base, CH), :], in_ca, sem_na)
    pltpu.async_copy(logits_hbm.at[pl.ds(base + CH, CH), :], in_cb, sem_nb)

    lane = lax.iota(jnp.int32, 16)
    lt8 = lane < 8
    zeros16 = jnp.zeros((16,), jnp.int32)
    for j in range(4):
        cnt_v[pl.ds(16 * j, 16)] = zeros16
    bias_regs = [bias_v[pl.ds(16 * j, 16)] for j in range(4)]
    ones16 = jnp.ones((16,), jnp.int32)

    def merge(k0, v0, k1, v1):
        k1r = lax.rev(k1, (0,))
        v1r = lax.rev(v1, (0,))
        km = jnp.where(lt8, k0, k1r)
        vm = jnp.where(lt8, v0, v1r)
        return plsc.sort_key_val(km, vm, descending=True)

    def take(x, idx):
        return x.at[idx].get(mode="promise_in_bounds")

    lane_and7 = jnp.bitwise_and(lane, 7)

    def merge_tree(in_c, r):
        sk = []
        sv = []
        for j in range(4):
            x = in_c[r, pl.ds(16 * j, 16)]
            sg = 1.0 / (1.0 + jnp.exp(-x))
            b = sg + bias_regs[j]
            k_s, v_s = plsc.sort_key_val(b, lane + 16 * j, descending=True)
            sk.append(k_s)
            sv.append(v_s)
        ka, va = merge(sk[0], sv[0], sk[1], sv[1])
        kb, vb = merge(sk[2], sv[2], sk[3], sv[3])
        return merge(ka, va, kb, vb)

    shr3 = lax.shift_right_logical(lane, 3)

    def do_pair(p, pbase, in_c, w_c, i_c):
        rA = 2 * (p - pbase)
        kA, vA = merge_tree(in_c, rA)
        kB, vB = merge_tree(in_c, rA + 1)
        kP = jnp.where(lt8, kA, take(kB, lane_and7))
        vP = jnp.where(lt8, vA, take(vB, lane_and7))
        bg = plsc.load_gather(bias_v, [vP])
        w = kP - bg
        s = w
        for m in (1, 2, 4):
            s = s + take(s, jnp.bitwise_xor(lane, m))
        wn = (w * SCALE) / (s + 1e-20)
        rowv = 2 * (p - pbase) + shr3
        plsc.store_scatter(w_c, [rowv, lane_and7], wn)
        plsc.store_scatter(i_c, [rowv, lane_and7], vP)
        plsc.addupdate_scatter(cnt_v, [vP], ones16, mask=lt8)
        plsc.addupdate_scatter(cnt_v, [vP], ones16, mask=jnp.logical_not(lt8))

    PPC = CH // 2

    def run_chunk(ch, ch2, in_c, w_c, i_c, sem_n, sem_w, sem_i):
        pltpu.make_async_copy(
            logits_hbm.at[pl.ds(base, CH), :], in_c, sem_n).wait()

        @pl.when(ch2 > 0)
        def _():
            pltpu.make_async_copy(
                w_c, w_hbm.at[pl.ds(base, CH), :], sem_w).wait()
            pltpu.make_async_copy(
                i_c, idx_hbm.at[pl.ds(base, CH), :], sem_i).wait()

        pbase = ch * PPC

        @plsc.parallel_loop(0, PPC, unroll=2)
        def _(p):
            do_pair(pbase + p, pbase, in_c, w_c, i_c)

        row0 = base + ch * CH
        pltpu.async_copy(w_c, w_hbm.at[pl.ds(row0, CH), :], sem_w)
        pltpu.async_copy(i_c, idx_hbm.at[pl.ds(row0, CH), :], sem_i)

        @pl.when(ch + 2 < NCH)
        def _():
            rown = base + (ch + 2) * CH
            pltpu.async_copy(logits_hbm.at[pl.ds(rown, CH), :], in_c, sem_n)

    def outer(ch2, carry):
        run_chunk(2 * ch2, ch2, in_ca, w_ca, i_ca, sem_na, sem_wa, sem_ia)
        run_chunk(2 * ch2 + 1, ch2, in_cb, w_cb, i_cb, sem_nb, sem_wb, sem_ib)
        return carry

    lax.fori_loop(0, NCH // 2, outer, 0)

    for w_c, i_c, sem_w, sem_i in ((w_ca, i_ca, sem_wa, sem_ia),
                                   (w_cb, i_cb, sem_wb, sem_ib)):
        pltpu.make_async_copy(w_c, w_hbm.at[pl.ds(base, CH), :], sem_w).wait()
        pltpu.make_async_copy(i_c, idx_hbm.at[pl.ds(base, CH), :], sem_i).wait()

    pltpu.sync_copy(cnt_v, cnt_hbm.at[pl.ds(wid * E, E)])


_router = pl.kernel(
    _router_body,
    out_type=(
        jax.ShapeDtypeStruct((T, TOPK), jnp.float32),
        jax.ShapeDtypeStruct((T, TOPK), jnp.int32),
        jax.ShapeDtypeStruct((NW * E,), jnp.int32),
    ),
    mesh=plsc.VectorSubcoreMesh(core_axis_name="c", subcore_axis_name="s"),
    compiler_params=pltpu.CompilerParams(needs_layout_passes=False,
                                         use_tc_tiling_on_sc=True),
    scratch_types=(
        pltpu.VMEM((CH, E), jnp.float32),
        pltpu.VMEM((CH, E), jnp.float32),
        pltpu.VMEM((CH, TOPK), jnp.float32),
        pltpu.VMEM((CH, TOPK), jnp.float32),
        pltpu.VMEM((CH, TOPK), jnp.int32),
        pltpu.VMEM((CH, TOPK), jnp.int32),
        pltpu.VMEM((E,), jnp.int32),
        pltpu.VMEM((E,), jnp.float32),
        pltpu.SemaphoreType.DMA,
        pltpu.SemaphoreType.DMA,
        pltpu.SemaphoreType.DMA,
        pltpu.SemaphoreType.DMA,
        pltpu.SemaphoreType.DMA,
        pltpu.SemaphoreType.DMA,
    ),
)


def kernel(logits, e_score_correction_bias):
    topk_weight, topk_idx, cnt_part = _router(logits,
                                              e_score_correction_bias)
    tokens_per_expert = cnt_part.reshape(NW, E).sum(axis=0)
    return (logits, topk_weight, topk_idx, tokens_per_expert)
